# baseline (speedup 1.0000x reference)
import functools

import jax
import jax.numpy as jnp
from jax import lax
from jax.experimental import pallas as pl
from jax.experimental.pallas import tpu as pltpu
from jax.experimental.pallas import tpu_sc as plsc

N = 10000
E = 160000
NC_CORES = 2
NS = 16
NW = NC_CORES * NS
C = 128
EP = E + C
RNG = 312
CORE_N = RNG * NS
CORE_N1 = N - CORE_N


def _matmul_kernel(x_ref, wk_ref, bk_ref, wqv_ref, bqv_ref, ws_ref, bs_ref,
                   k_ref, qv_ref, s_ref):
    xb = x_ref[...]
    hp = jax.lax.Precision.HIGHEST
    k_ref[0] = jnp.dot(xb, wk_ref[0], precision=hp,
                       preferred_element_type=jnp.float32) + bk_ref[0]
    qv_ref[0] = jnp.dot(xb, wqv_ref[0], precision=hp,
                        preferred_element_type=jnp.float32) + bqv_ref[0]
    s_ref[...] = jnp.dot(xb, ws_ref[0], precision=hp,
                         preferred_element_type=jnp.float32) + bs_ref[0]


@functools.lru_cache(maxsize=None)
def _make_matmul(din, dout, nb=1000):
    nc = dout // 128
    grid = (nc, N // nb)
    return pl.pallas_call(
        _matmul_kernel,
        grid=grid,
        in_specs=[
            pl.BlockSpec((nb, din), lambda c, i: (i, 0)),
            pl.BlockSpec((1, din, 128), lambda c, i: (c, 0, 0)),
            pl.BlockSpec((1, 1, 128), lambda c, i: (c, 0, 0)),
            pl.BlockSpec((1, din, 256), lambda c, i: (c, 0, 0)),
            pl.BlockSpec((1, 1, 256), lambda c, i: (c, 0, 0)),
            pl.BlockSpec((1, din, 128), lambda c, i: (c, 0, 0)),
            pl.BlockSpec((1, 1, 128), lambda c, i: (c, 0, 0)),
        ],
        out_specs=[
            pl.BlockSpec((1, nb, 128), lambda c, i: (c, i, 0)),
            pl.BlockSpec((1, nb, 256), lambda c, i: (c, i, 0)),
            pl.BlockSpec((nb, 128), lambda c, i: (i, c)),
        ],
        out_shape=[
            jax.ShapeDtypeStruct((nc, N, 128), jnp.float32),
            jax.ShapeDtypeStruct((nc, N, 256), jnp.float32),
            jax.ShapeDtypeStruct((N, dout), jnp.float32),
        ],
    )


def _combine_kernel(p_ref, s_ref, o_ref, *, relu):
    o = p_ref[0] + s_ref[...]
    if relu:
        o = jnp.maximum(o, 0.0)
    o_ref[...] = o


@functools.lru_cache(maxsize=None)
def _make_combine(dout, relu, nb=1000):
    nc = dout // 128
    grid = (nc, N // nb)
    return pl.pallas_call(
        functools.partial(_combine_kernel, relu=relu),
        grid=grid,
        in_specs=[
            pl.BlockSpec((1, nb, 128), lambda c, i: (c, i, 0)),
            pl.BlockSpec((nb, 128), lambda c, i: (i, c)),
        ],
        out_specs=pl.BlockSpec((nb, 128), lambda c, i: (i, c)),
        out_shape=jax.ShapeDtypeStruct((N, dout), jnp.float32),
    )


def _zero_rows(ref, lo, hi):
    @plsc.parallel_loop(lo, hi, unroll=2)
    def body(r):
        for f in range(8):
            ref[r, pl.ds(f * 16, 16)] = jnp.zeros((16,), jnp.float32)


@functools.lru_cache(maxsize=None)
def _make_edge(nc):
    mesh = plsc.VectorSubcoreMesh(core_axis_name="c", subcore_axis_name="s")

    @functools.partial(
        pl.kernel,
        out_type=jax.ShapeDtypeStruct((nc, N, 128), jnp.float32),
        mesh=mesh,
        scratch_types=[
            pltpu.VMEM((48,), jnp.int32),
            pltpu.VMEM((8, C), jnp.int32),
            pltpu.VMEM((8, C), jnp.int32),
            pltpu.VMEM((8, C), jnp.int32),
            pltpu.VMEM((C, 128), jnp.float32),
            pltpu.VMEM((C, 256), jnp.float32),
            pltpu.VMEM((C, 128), jnp.float32),
            pltpu.VMEM_SHARED((CORE_N1, 128), jnp.float32),
            pltpu.SemaphoreType.DMA,
        ],
    )
    def edge_kernel(k_hbm, qv_hbm, src_hbm, dst_hbm, bnd_hbm, out_hbm,
                    bnd_vm, src_v, dst_v, dadj_v, kd_v, qv_v, msg_v,
                    agg_sh, sem):
        ci = lax.axis_index("c")
        si = lax.axis_index("s")
        w = ci * NS + si
        pltpu.sync_copy(bnd_hbm, bnd_vm)
        b_lo = bnd_vm[pl.ds(w, 16)][0]
        b_hi = bnd_vm[pl.ds(w + 1, 16)][0]
        e0 = (b_lo // 8) * 8
        nchw = (b_hi - e0 + C - 1) // C
        core_base = ci * CORE_N
        lbase = si * RNG

        for ch in range(nc):
            _zero_rows(msg_v, 0, C)
            pltpu.sync_copy(msg_v, agg_sh.at[pl.ds(lbase, 128)])
            pltpu.sync_copy(msg_v, agg_sh.at[pl.ds(lbase + 128, 128)])

            @pl.when(si < NS - 1)
            def _ztail_a():
                pltpu.sync_copy(msg_v.at[pl.ds(0, 56)],
                                agg_sh.at[pl.ds(lbase + 256, 56)])

            @pl.when(si == NS - 1)
            def _ztail_b():
                pltpu.sync_copy(msg_v.at[pl.ds(0, 72)],
                                agg_sh.at[pl.ds(lbase + 256, 72)])

            def echunk(j, _):
                cbase = e0 + j * C
                pltpu.sync_copy(src_hbm.at[pl.ds(cbase, C)], src_v.at[0])
                pltpu.sync_copy(dst_hbm.at[pl.ds(cbase, C)], dst_v.at[0])
                cp1 = pltpu.async_copy(k_hbm.at[ch].at[dst_v.at[0]], kd_v, sem)
                cp2 = pltpu.async_copy(qv_hbm.at[ch].at[src_v.at[0]], qv_v, sem)
                for g in range(8):
                    d = dst_v[0, pl.ds(g * 16, 16)]
                    dl = d - core_base
                    dl = jnp.minimum(jnp.maximum(dl, 0), CORE_N1 - 1)
                    dadj_v[0, pl.ds(g * 16, 16)] = dl
                cp1.wait()
                cp2.wait()

                @plsc.parallel_loop(0, C, unroll=4)
                def rbody(r):
                    for f in range(8):
                        kd = kd_v[r, pl.ds(f * 16, 16)]
                        qq = qv_v[r, pl.ds(f * 16, 16)]
                        vv = qv_v[r, pl.ds(128 + f * 16, 16)]
                        g = 1.0 / (1.0 + jnp.exp(-(kd + qq)))
                        msg_v[r, pl.ds(f * 16, 16)] = g * vv

                @pl.when(j == 0)
                def _head():
                    _zero_rows(msg_v, 0, b_lo - e0)

                @pl.when(j == nchw - 1)
                def _tail():
                    _zero_rows(msg_v, b_hi - cbase, C)

                pltpu.sync_copy(msg_v, agg_sh.at[dadj_v.at[0]], add=True)
                return _
            lax.fori_loop(0, nchw, echunk, None)

            @pl.when(w < NW - 1)
            def _wb_a():
                pltpu.sync_copy(
                    agg_sh.at[pl.ds(lbase, RNG)],
                    out_hbm.at[ch].at[pl.ds(core_base + lbase, RNG)])

            @pl.when(w == NW - 1)
            def _wb_b():
                pltpu.sync_copy(
                    agg_sh.at[pl.ds(lbase, RNG + 16)],
                    out_hbm.at[ch].at[pl.ds(core_base + lbase, RNG + 16)])

    return edge_kernel


DIMS = ((256, 512), (512, 512), (512, 256))


def kernel(x, edge_index, edge_type,
           W_key0, b_key0, W_query0, b_query0, W_value0, b_value0, W_skip0, b_skip0,
           W_key1, b_key1, W_query1, b_query1, W_value1, b_value1, W_skip1, b_skip1,
           W_key2, b_key2, W_query2, b_query2, W_value2, b_value2, W_skip2, b_skip2):
    params = (W_key0, b_key0, W_query0, b_query0, W_value0, b_value0, W_skip0, b_skip0,
              W_key1, b_key1, W_query1, b_query1, W_value1, b_value1, W_skip1, b_skip1,
              W_key2, b_key2, W_query2, b_query2, W_value2, b_value2, W_skip2, b_skip2)
    src = edge_index[0].astype(jnp.int32)
    dst = edge_index[1].astype(jnp.int32)
    perm = jnp.argsort(dst)
    dst_s = jnp.pad(dst[perm], (0, EP - E))
    src_s = jnp.pad(src[perm], (0, EP - E))
    starts = jnp.arange(NW, dtype=jnp.int32) * RNG
    bnd = jnp.searchsorted(dst_s[:E], starts, side="left").astype(jnp.int32)
    bnd = jnp.concatenate([bnd, jnp.full((16,), E, jnp.int32)])

    h = x
    for l, (din, dout) in enumerate(DIMS):
        Wk, bk, Wq, bq, Wv, bv, Ws, bs = params[l * 8:(l + 1) * 8]
        nc = dout // 128
        WK = Wk.reshape(nc, 128, din).transpose(0, 2, 1)
        WQV = jnp.concatenate(
            [Wq.reshape(nc, 128, din), Wv.reshape(nc, 128, din)],
            axis=1).transpose(0, 2, 1)
        WS = Ws.reshape(nc, 128, din).transpose(0, 2, 1)
        bK = bk.reshape(nc, 1, 128)
        bQV = jnp.concatenate([bq.reshape(nc, 1, 128), bv.reshape(nc, 1, 128)],
                              axis=2)
        bS = bs.reshape(nc, 1, 128)

        K, QV, S = _make_matmul(din, dout)(h, WK, bK, WQV, bQV, WS, bS)
        P = _make_edge(nc)(K, QV, src_s, dst_s, bnd)
        h = _make_combine(dout, l < 2)(P, S)
    return h


# speedup vs baseline: 1.4940x; 1.4940x over previous
import functools

import jax
import jax.numpy as jnp
from jax import lax
from jax.experimental import pallas as pl
from jax.experimental.pallas import tpu as pltpu
from jax.experimental.pallas import tpu_sc as plsc

N = 10000
E = 160000
NC_CORES = 2
NS = 16
NW = NC_CORES * NS
C = 96
MCH = E // C + 2
EP = MCH * C
RNG = 312
CORE_N = RNG * NS
CORE_N1 = N - CORE_N


def _matmul_kernel(x_ref, wk_ref, bk_ref, wqv_ref, bqv_ref, ws_ref, bs_ref,
                   k_ref, qv_ref, s_ref):
    xb = x_ref[...]
    hp = jax.lax.Precision.HIGHEST
    k_ref[0] = jnp.dot(xb, wk_ref[0], precision=hp,
                       preferred_element_type=jnp.float32) + bk_ref[0]
    qv_ref[0] = jnp.dot(xb, wqv_ref[0], precision=hp,
                        preferred_element_type=jnp.float32) + bqv_ref[0]
    s_ref[...] = jnp.dot(xb, ws_ref[0], precision=hp,
                         preferred_element_type=jnp.float32) + bs_ref[0]


@functools.lru_cache(maxsize=None)
def _make_matmul(din, dout, nb=1000):
    nc = dout // 128
    grid = (nc, N // nb)
    return pl.pallas_call(
        _matmul_kernel,
        grid=grid,
        in_specs=[
            pl.BlockSpec((nb, din), lambda c, i: (i, 0)),
            pl.BlockSpec((1, din, 128), lambda c, i: (c, 0, 0)),
            pl.BlockSpec((1, 1, 128), lambda c, i: (c, 0, 0)),
            pl.BlockSpec((1, din, 256), lambda c, i: (c, 0, 0)),
            pl.BlockSpec((1, 1, 256), lambda c, i: (c, 0, 0)),
            pl.BlockSpec((1, din, 128), lambda c, i: (c, 0, 0)),
            pl.BlockSpec((1, 1, 128), lambda c, i: (c, 0, 0)),
        ],
        out_specs=[
            pl.BlockSpec((1, nb, 128), lambda c, i: (c, i, 0)),
            pl.BlockSpec((1, nb, 256), lambda c, i: (c, i, 0)),
            pl.BlockSpec((nb, 128), lambda c, i: (i, c)),
        ],
        out_shape=[
            jax.ShapeDtypeStruct((nc, N, 128), jnp.float32),
            jax.ShapeDtypeStruct((nc, N, 256), jnp.float32),
            jax.ShapeDtypeStruct((N, dout), jnp.float32),
        ],
    )


def _combine_kernel(p_ref, s_ref, o_ref, *, relu):
    o = p_ref[0] + s_ref[...]
    if relu:
        o = jnp.maximum(o, 0.0)
    o_ref[...] = o


@functools.lru_cache(maxsize=None)
def _make_combine(dout, relu, nb=1000):
    nc = dout // 128
    grid = (nc, N // nb)
    return pl.pallas_call(
        functools.partial(_combine_kernel, relu=relu),
        grid=grid,
        in_specs=[
            pl.BlockSpec((1, nb, 128), lambda c, i: (c, i, 0)),
            pl.BlockSpec((nb, 128), lambda c, i: (i, c)),
        ],
        out_specs=pl.BlockSpec((nb, 128), lambda c, i: (i, c)),
        out_shape=jax.ShapeDtypeStruct((N, dout), jnp.float32),
    )


def _zero_rows(ref, lo, hi):
    @plsc.parallel_loop(lo, hi, unroll=2)
    def body(r):
        for f in range(8):
            ref[r, pl.ds(f * 16, 16)] = jnp.zeros((16,), jnp.float32)


@functools.lru_cache(maxsize=None)
def _make_edge(nc):
    mesh = plsc.VectorSubcoreMesh(core_axis_name="c", subcore_axis_name="s")

    @functools.partial(
        pl.kernel,
        out_type=jax.ShapeDtypeStruct((nc, N, 128), jnp.float32),
        mesh=mesh,
        scratch_types=[
            pltpu.VMEM((48,), jnp.int32),
            pltpu.VMEM((2, C), jnp.int32),
            pltpu.VMEM((2, C), jnp.int32),
            pltpu.VMEM((8, C), jnp.int32),
            pltpu.VMEM((C, 128), jnp.float32),
            pltpu.VMEM((C, 128), jnp.float32),
            pltpu.VMEM((C, 256), jnp.float32),
            pltpu.VMEM((C, 256), jnp.float32),
            pltpu.VMEM((C, 128), jnp.float32),
            pltpu.VMEM_SHARED((CORE_N1, 128), jnp.float32),
            pltpu.SemaphoreType.DMA,
            pltpu.SemaphoreType.DMA,
            pltpu.SemaphoreType.DMA,
        ],
    )
    def edge_kernel(k_hbm, qv_hbm, sd_hbm, bnd_hbm, out_hbm,
                    bnd_vm, ib0, ib1, dadj_v, kd0, kd1, qv0, qv1, msg_v,
                    agg_sh, sg0, sg1, sems):
        ci = lax.axis_index("c")
        si = lax.axis_index("s")
        w = ci * NS + si
        pltpu.sync_copy(bnd_hbm, bnd_vm)
        b_lo = bnd_vm[pl.ds(w, 16)][0]
        b_hi = bnd_vm[pl.ds(w + 1, 16)][0]
        m_lo = b_lo // C
        ncnt = (b_hi + C - 1) // C - m_lo
        core_base = ci * CORE_N
        lbase = si * RNG
        IB = (ib0, ib1)
        KD = (kd0, kd1)
        QVB = (qv0, qv1)
        SG = (sg0, sg1)

        def fchunk(ch, _c):
            _zero_rows(msg_v, 0, C)
            for t in range(3):
                pltpu.sync_copy(msg_v, agg_sh.at[pl.ds(lbase + 96 * t, 96)])

            @pl.when(si < NS - 1)
            def _ztail_a():
                pltpu.sync_copy(msg_v.at[pl.ds(0, 24)],
                                agg_sh.at[pl.ds(lbase + 288, 24)])

            @pl.when(si == NS - 1)
            def _ztail_b():
                pltpu.sync_copy(msg_v.at[pl.ds(0, 40)],
                                agg_sh.at[pl.ds(lbase + 288, 40)])

            @pl.when(ncnt > 0)
            def _prologue():
                pltpu.sync_copy(sd_hbm.at[m_lo], ib0)
                pltpu.async_copy(k_hbm.at[ch].at[ib0.at[0]], kd0, sg0)
                pltpu.async_copy(qv_hbm.at[ch].at[ib0.at[1]], qv0, sg0)

            def epair(jj, _):
                for p in (0, 1):
                    m = 2 * jj + p
                    mabs = m_lo + m
                    ib, kd_v, qv_v, sg = IB[p], KD[p], QVB[p], SG[p]

                    @pl.when(m < ncnt)
                    def _step():
                        @pl.when(m + 1 < ncnt)
                        def _prefetch():
                            pltpu.sync_copy(sd_hbm.at[mabs + 1], IB[1 - p])
                            pltpu.async_copy(
                                k_hbm.at[ch].at[IB[1 - p].at[0]],
                                KD[1 - p], SG[1 - p])
                            pltpu.async_copy(
                                qv_hbm.at[ch].at[IB[1 - p].at[1]],
                                QVB[1 - p], SG[1 - p])

                        @pl.when(m >= 1)
                        def _drain():
                            pltpu.make_async_copy(
                                msg_v, agg_sh.at[dadj_v.at[0]], sems).wait()

                        for g in range(C // 16):
                            d = ib[0, pl.ds(g * 16, 16)]
                            dl = d - core_base
                            dl = jnp.minimum(jnp.maximum(dl, 0), CORE_N1 - 1)
                            dadj_v[0, pl.ds(g * 16, 16)] = dl

                        pltpu.make_async_copy(
                            k_hbm.at[ch].at[ib.at[0]], kd_v, sg).wait()
                        pltpu.make_async_copy(
                            qv_hbm.at[ch].at[ib.at[1]], qv_v, sg).wait()

                        @plsc.parallel_loop(0, C, unroll=4)
                        def rbody(r):
                            for f in range(8):
                                kd = kd_v[r, pl.ds(f * 16, 16)]
                                qq = qv_v[r, pl.ds(f * 16, 16)]
                                vv = qv_v[r, pl.ds(128 + f * 16, 16)]
                                g = 1.0 / (1.0 + jnp.exp(-(kd + qq)))
                                msg_v[r, pl.ds(f * 16, 16)] = g * vv

                        lo_r = jnp.minimum(jnp.maximum(b_lo - mabs * C, 0), C)
                        hi_r = jnp.minimum(jnp.maximum(b_hi - mabs * C, 0), C)
                        _zero_rows(msg_v, 0, lo_r)
                        _zero_rows(msg_v, hi_r, C)

                        cp = pltpu.async_copy(
                            msg_v, agg_sh.at[dadj_v.at[0]], sems, add=True)

                        @pl.when(m == ncnt - 1)
                        def _last():
                            cp.wait()
                return _
            lax.fori_loop(0, (ncnt + 1) // 2, epair, None)

            @pl.when(w < NW - 1)
            def _wb_a():
                pltpu.sync_copy(
                    agg_sh.at[pl.ds(lbase, RNG)],
                    out_hbm.at[ch].at[pl.ds(core_base + lbase, RNG)])

            @pl.when(w == NW - 1)
            def _wb_b():
                pltpu.sync_copy(
                    agg_sh.at[pl.ds(lbase, RNG + 16)],
                    out_hbm.at[ch].at[pl.ds(core_base + lbase, RNG + 16)])
            return _c
        lax.fori_loop(0, nc, fchunk, None)

    return edge_kernel


DIMS = ((256, 512), (512, 512), (512, 256))


def kernel(x, edge_index, edge_type,
           W_key0, b_key0, W_query0, b_query0, W_value0, b_value0, W_skip0, b_skip0,
           W_key1, b_key1, W_query1, b_query1, W_value1, b_value1, W_skip1, b_skip1,
           W_key2, b_key2, W_query2, b_query2, W_value2, b_value2, W_skip2, b_skip2):
    params = (W_key0, b_key0, W_query0, b_query0, W_value0, b_value0, W_skip0, b_skip0,
              W_key1, b_key1, W_query1, b_query1, W_value1, b_value1, W_skip1, b_skip1,
              W_key2, b_key2, W_query2, b_query2, W_value2, b_value2, W_skip2, b_skip2)
    src = edge_index[0].astype(jnp.int32)
    dst = edge_index[1].astype(jnp.int32)
    perm = jnp.argsort(dst)
    dst_s = jnp.pad(dst[perm], (0, EP - E))
    src_s = jnp.pad(src[perm], (0, EP - E))
    sd = jnp.stack([dst_s.reshape(MCH, C), src_s.reshape(MCH, C)], axis=1)
    starts = jnp.arange(NW, dtype=jnp.int32) * RNG
    bnd = jnp.searchsorted(dst_s[:E], starts, side="left").astype(jnp.int32)
    bnd = jnp.concatenate([bnd, jnp.full((16,), E, jnp.int32)])

    h = x
    for l, (din, dout) in enumerate(DIMS):
        Wk, bk, Wq, bq, Wv, bv, Ws, bs = params[l * 8:(l + 1) * 8]
        nc = dout // 128
        WK = Wk.reshape(nc, 128, din).transpose(0, 2, 1)
        WQV = jnp.concatenate(
            [Wq.reshape(nc, 128, din), Wv.reshape(nc, 128, din)],
            axis=1).transpose(0, 2, 1)
        WS = Ws.reshape(nc, 128, din).transpose(0, 2, 1)
        bK = bk.reshape(nc, 1, 128)
        bQV = jnp.concatenate([bq.reshape(nc, 1, 128), bv.reshape(nc, 1, 128)],
                              axis=2)
        bS = bs.reshape(nc, 1, 128)

        K, QV, S = _make_matmul(din, dout)(h, WK, bK, WQV, bQV, WS, bS)
        P = _make_edge(nc)(K, QV, sd, bnd)
        h = _make_combine(dout, l < 2)(P, S)
    return h


# speedup vs baseline: 1.5154x; 1.0143x over previous
import functools

import jax
import jax.numpy as jnp
from jax import lax
from jax.experimental import pallas as pl
from jax.experimental.pallas import tpu as pltpu
from jax.experimental.pallas import tpu_sc as plsc

N = 10000
E = 160000
NC_CORES = 2
NS = 16
NW = NC_CORES * NS
C = 96
MCH = E // C + 2
EP = MCH * C
RNG = 312
CORE_N = RNG * NS
CORE_N1 = N - CORE_N


def _matmul_kernel(x_ref, wk_ref, bk_ref, wqv_ref, bqv_ref, ws_ref, bs_ref,
                   k_ref, qv_ref, s_ref):
    xb = x_ref[...]
    hp = jax.lax.Precision.HIGHEST
    k_ref[0] = jnp.dot(xb, wk_ref[0], precision=hp,
                       preferred_element_type=jnp.float32) + bk_ref[0]
    qv_ref[0] = jnp.dot(xb, wqv_ref[0], precision=hp,
                        preferred_element_type=jnp.float32) + bqv_ref[0]
    s_ref[...] = jnp.dot(xb, ws_ref[0], precision=hp,
                         preferred_element_type=jnp.float32) + bs_ref[0]


@functools.lru_cache(maxsize=None)
def _make_matmul(din, dout, nb=1000):
    nc = dout // 128
    grid = (nc, N // nb)
    return pl.pallas_call(
        _matmul_kernel,
        grid=grid,
        in_specs=[
            pl.BlockSpec((nb, din), lambda c, i: (i, 0)),
            pl.BlockSpec((1, din, 128), lambda c, i: (c, 0, 0)),
            pl.BlockSpec((1, 1, 128), lambda c, i: (c, 0, 0)),
            pl.BlockSpec((1, din, 256), lambda c, i: (c, 0, 0)),
            pl.BlockSpec((1, 1, 256), lambda c, i: (c, 0, 0)),
            pl.BlockSpec((1, din, 128), lambda c, i: (c, 0, 0)),
            pl.BlockSpec((1, 1, 128), lambda c, i: (c, 0, 0)),
        ],
        out_specs=[
            pl.BlockSpec((1, nb, 128), lambda c, i: (c, i, 0)),
            pl.BlockSpec((1, nb, 256), lambda c, i: (c, i, 0)),
            pl.BlockSpec((nb, 128), lambda c, i: (i, c)),
        ],
        out_shape=[
            jax.ShapeDtypeStruct((nc, N, 128), jnp.float32),
            jax.ShapeDtypeStruct((nc, N, 256), jnp.float32),
            jax.ShapeDtypeStruct((N, dout), jnp.float32),
        ],
    )


def _combine_kernel(p_ref, s_ref, o_ref, *, relu):
    o = p_ref[0] + s_ref[...]
    if relu:
        o = jnp.maximum(o, 0.0)
    o_ref[...] = o


@functools.lru_cache(maxsize=None)
def _make_combine(dout, relu, nb=1000):
    nc = dout // 128
    grid = (nc, N // nb)
    return pl.pallas_call(
        functools.partial(_combine_kernel, relu=relu),
        grid=grid,
        in_specs=[
            pl.BlockSpec((1, nb, 128), lambda c, i: (c, i, 0)),
            pl.BlockSpec((nb, 128), lambda c, i: (i, c)),
        ],
        out_specs=pl.BlockSpec((nb, 128), lambda c, i: (i, c)),
        out_shape=jax.ShapeDtypeStruct((N, dout), jnp.float32),
    )


def _zero_rows(ref, lo, hi):
    @plsc.parallel_loop(lo, hi, unroll=2)
    def body(r):
        for f in range(8):
            ref[r, pl.ds(f * 16, 16)] = jnp.zeros((16,), jnp.float32)


@functools.lru_cache(maxsize=None)
def _make_edge(nc):
    mesh = plsc.VectorSubcoreMesh(core_axis_name="c", subcore_axis_name="s")

    @functools.partial(
        pl.kernel,
        out_type=jax.ShapeDtypeStruct((nc, N, 128), jnp.float32),
        mesh=mesh,
        scratch_types=[
            pltpu.VMEM((48,), jnp.int32),
            pltpu.VMEM((2, C), jnp.int32),
            pltpu.VMEM((2, C), jnp.int32),
            pltpu.VMEM((8, C), jnp.int32),
            pltpu.VMEM((C, 128), jnp.float32),
            pltpu.VMEM((C, 128), jnp.float32),
            pltpu.VMEM((C, 256), jnp.float32),
            pltpu.VMEM((C, 256), jnp.float32),
            pltpu.VMEM((C, 128), jnp.float32),
            pltpu.VMEM_SHARED((CORE_N1, 128), jnp.float32),
            pltpu.SemaphoreType.DMA,
            pltpu.SemaphoreType.DMA,
            pltpu.SemaphoreType.DMA,
        ],
    )
    def edge_kernel(k_hbm, qv_hbm, sd_hbm, bnd_hbm, out_hbm,
                    bnd_vm, ib0, ib1, dadj_v, kd0, kd1, qv0, qv1, msg_v,
                    agg_sh, sg0, sg1, sems):
        ci = lax.axis_index("c")
        si = lax.axis_index("s")
        w = ci * NS + si
        pltpu.sync_copy(bnd_hbm, bnd_vm)
        b_lo = bnd_vm[pl.ds(w, 16)][0]
        b_hi = bnd_vm[pl.ds(w + 1, 16)][0]
        m_lo = b_lo // C
        ncnt = (b_hi + C - 1) // C - m_lo
        core_base = ci * CORE_N
        lbase = si * RNG
        IB = (ib0, ib1)
        KD = (kd0, kd1)
        QVB = (qv0, qv1)
        SG = (sg0, sg1)

        def fchunk(ch, _c):
            _zero_rows(msg_v, 0, C)
            for t in range(3):
                pltpu.sync_copy(msg_v, agg_sh.at[pl.ds(lbase + 96 * t, 96)])

            @pl.when(si < NS - 1)
            def _ztail_a():
                pltpu.sync_copy(msg_v.at[pl.ds(0, 24)],
                                agg_sh.at[pl.ds(lbase + 288, 24)])

            @pl.when(si == NS - 1)
            def _ztail_b():
                pltpu.sync_copy(msg_v.at[pl.ds(0, 40)],
                                agg_sh.at[pl.ds(lbase + 288, 40)])

            @pl.when(ncnt > 0)
            def _prologue():
                pltpu.sync_copy(sd_hbm.at[m_lo], ib0)
                pltpu.async_copy(k_hbm.at[ch].at[ib0.at[0]], kd0, sg0)
                pltpu.async_copy(qv_hbm.at[ch].at[ib0.at[1]], qv0, sg0)

            def epair(jj, _):
                for p in (0, 1):
                    m = 2 * jj + p
                    mabs = m_lo + m
                    ib, kd_v, qv_v, sg = IB[p], KD[p], QVB[p], SG[p]

                    @pl.when(m < ncnt)
                    def _step():
                        @pl.when(m + 1 < ncnt)
                        def _prefetch():
                            pltpu.sync_copy(sd_hbm.at[mabs + 1], IB[1 - p])
                            pltpu.async_copy(
                                k_hbm.at[ch].at[IB[1 - p].at[0]],
                                KD[1 - p], SG[1 - p])
                            pltpu.async_copy(
                                qv_hbm.at[ch].at[IB[1 - p].at[1]],
                                QVB[1 - p], SG[1 - p])

                        pltpu.make_async_copy(
                            k_hbm.at[ch].at[ib.at[0]], kd_v, sg).wait()
                        pltpu.make_async_copy(
                            qv_hbm.at[ch].at[ib.at[1]], qv_v, sg).wait()

                        @pl.when(m >= 1)
                        def _drain():
                            pltpu.make_async_copy(
                                msg_v, agg_sh.at[dadj_v.at[0]], sems).wait()

                        for g in range(C // 16):
                            d = ib[0, pl.ds(g * 16, 16)]
                            dl = d - core_base
                            dl = jnp.minimum(jnp.maximum(dl, 0), CORE_N1 - 1)
                            dadj_v[0, pl.ds(g * 16, 16)] = dl

                        @plsc.parallel_loop(0, C, unroll=6)
                        def rbody(r):
                            for f in range(8):
                                kd = kd_v[r, pl.ds(f * 16, 16)]
                                qq = qv_v[r, pl.ds(f * 16, 16)]
                                vv = qv_v[r, pl.ds(128 + f * 16, 16)]
                                g = 1.0 / (1.0 + jnp.exp(-(kd + qq)))
                                msg_v[r, pl.ds(f * 16, 16)] = g * vv

                        lo_r = jnp.minimum(jnp.maximum(b_lo - mabs * C, 0), C)
                        hi_r = jnp.minimum(jnp.maximum(b_hi - mabs * C, 0), C)
                        _zero_rows(msg_v, 0, lo_r)
                        _zero_rows(msg_v, hi_r, C)

                        cp = pltpu.async_copy(
                            msg_v, agg_sh.at[dadj_v.at[0]], sems, add=True)

                        @pl.when(m == ncnt - 1)
                        def _last():
                            cp.wait()
                return _
            lax.fori_loop(0, (ncnt + 1) // 2, epair, None)

            @pl.when(w < NW - 1)
            def _wb_a():
                pltpu.sync_copy(
                    agg_sh.at[pl.ds(lbase, RNG)],
                    out_hbm.at[ch].at[pl.ds(core_base + lbase, RNG)])

            @pl.when(w == NW - 1)
            def _wb_b():
                pltpu.sync_copy(
                    agg_sh.at[pl.ds(lbase, RNG + 16)],
                    out_hbm.at[ch].at[pl.ds(core_base + lbase, RNG + 16)])
            return _c
        lax.fori_loop(0, nc, fchunk, None)

    return edge_kernel


DIMS = ((256, 512), (512, 512), (512, 256))


def kernel(x, edge_index, edge_type,
           W_key0, b_key0, W_query0, b_query0, W_value0, b_value0, W_skip0, b_skip0,
           W_key1, b_key1, W_query1, b_query1, W_value1, b_value1, W_skip1, b_skip1,
           W_key2, b_key2, W_query2, b_query2, W_value2, b_value2, W_skip2, b_skip2):
    params = (W_key0, b_key0, W_query0, b_query0, W_value0, b_value0, W_skip0, b_skip0,
              W_key1, b_key1, W_query1, b_query1, W_value1, b_value1, W_skip1, b_skip1,
              W_key2, b_key2, W_query2, b_query2, W_value2, b_value2, W_skip2, b_skip2)
    src = edge_index[0].astype(jnp.int32)
    dst = edge_index[1].astype(jnp.int32)
    perm = jnp.argsort(dst)
    dst_s = jnp.pad(dst[perm], (0, EP - E))
    src_s = jnp.pad(src[perm], (0, EP - E))
    sd = jnp.stack([dst_s.reshape(MCH, C), src_s.reshape(MCH, C)], axis=1)
    starts = jnp.arange(NW, dtype=jnp.int32) * RNG
    bnd = jnp.searchsorted(dst_s[:E], starts, side="left").astype(jnp.int32)
    bnd = jnp.concatenate([bnd, jnp.full((16,), E, jnp.int32)])

    h = x
    for l, (din, dout) in enumerate(DIMS):
        Wk, bk, Wq, bq, Wv, bv, Ws, bs = params[l * 8:(l + 1) * 8]
        nc = dout // 128
        WK = Wk.reshape(nc, 128, din).transpose(0, 2, 1)
        WQV = jnp.concatenate(
            [Wq.reshape(nc, 128, din), Wv.reshape(nc, 128, din)],
            axis=1).transpose(0, 2, 1)
        WS = Ws.reshape(nc, 128, din).transpose(0, 2, 1)
        bK = bk.reshape(nc, 1, 128)
        bQV = jnp.concatenate([bq.reshape(nc, 1, 128), bv.reshape(nc, 1, 128)],
                              axis=2)
        bS = bs.reshape(nc, 1, 128)

        K, QV, S = _make_matmul(din, dout)(h, WK, bK, WQV, bQV, WS, bS)
        P = _make_edge(nc)(K, QV, sd, bnd)
        h = _make_combine(dout, l < 2)(P, S)
    return h
